# baseline (speedup 1.0000x reference)
import functools

import jax
import jax.numpy as jnp
from jax import lax
from jax.experimental import pallas as pl
from jax.experimental.pallas import tpu as pltpu
from jax.experimental.pallas import tpu_sc as plsc

N = 50000
NP = 50176
D = 64
L = 2
E2 = 800000
E3 = 200000
NCORE = 2
NSUB = 16
NW = NCORE * NSUB

K2 = 32
K3 = 32
KS = 32
NB2 = E2 // K2
NB3 = E3 // K3
HALF = N // 2
HPAD = 25088
DUMB = HALF
ZR = 56
TW = 4 * D

_MESH = plsc.VectorSubcoreMesh(
    core_axis_name="c", subcore_axis_name="s", num_cores=NCORE, num_subcores=NSUB)

_ROWS_W = NP // NW
_GB = 128
_GNB = 12
_GT = _ROWS_W - _GNB * _GB


def _emb_body(xpad, emb, out, idxb, rows, idxt, rowst, sem):
    c = lax.axis_index("c")
    s = lax.axis_index("s")
    w = s * NCORE + c
    base = w * _ROWS_W
    @pl.loop(0, _GNB)
    def _(k):
        off = base + k * _GB
        pltpu.sync_copy(xpad.at[pl.ds(off, _GB)], idxb)
        pltpu.async_copy(emb.at[idxb], rows, sem).wait()
        pltpu.sync_copy(rows, out.at[pl.ds(off, _GB)])
    offt = base + _GNB * _GB
    pltpu.sync_copy(xpad.at[pl.ds(offt, _GT)], idxt)
    pltpu.async_copy(emb.at[idxt], rowst, sem).wait()
    pltpu.sync_copy(rowst, out.at[pl.ds(offt, _GT)])


def _emb_gather(xpad, embp):
    return pl.kernel(
        _emb_body,
        out_type=jax.ShapeDtypeStruct((NP, 2 * D), jnp.float32),
        mesh=_MESH,
        scratch_types=[
            pltpu.VMEM((_GB,), jnp.int32),
            pltpu.VMEM((_GB, 2 * D), jnp.float32),
            pltpu.VMEM((_GT,), jnp.int32),
            pltpu.VMEM((_GT, 2 * D), jnp.float32),
            pltpu.SemaphoreType.DMA,
        ],
    )(xpad, embp)


def _msg_body(e0, e1, t0, t1, t2, tb0, tb1, tt0, tt1, tt2,
              m0, m1, mt0, mt1, mt2,
              bidx, bmb, bmsg, tidx, tmb, tmsg, gsem, ssem):
    c = lax.axis_index("c")
    s = lax.axis_index("s")
    w = s * NCORE + c

    eb = (e0, e1)
    tabs = (tb0, tb1)
    mouts = (m0, m1)
    nI2 = 2 * -(-NB2 // (2 * NW))

    def b_issue(st, m):
        g = jnp.minimum(m * NW + w, NB2 - 1)
        off = g * K2
        for p in range(2):
            pltpu.sync_copy(eb[p].at[pl.ds(off, K2)], bidx[st][p])
        for p in range(2):
            pltpu.async_copy(tabs[p].at[bidx[st][p]], bmb[st][p], gsem[st][p])

    for st in range(2):
        b_issue(st, st)

    @pl.loop(0, nI2, step=2)
    def _(i):
        for st in range(2):
            m = i + st
            g = jnp.minimum(m * NW + w, NB2 - 1)
            for p in range(2):
                pltpu.make_async_copy(tabs[p].at[bidx[st][p]], bmb[st][p],
                                      gsem[st][p]).wait()
            @pl.loop(0, K2, unroll=4)
            def _(r):
                for q in range(8):
                    a = bmb[st][0][r, pl.ds(q * 16, 16)]
                    b = bmb[st][1][r, pl.ds(q * 16, 16)]
                    v = jnp.maximum(a + b, 0.0)
                    if q < 4:
                        bmsg[st][0][r, pl.ds(q * 16, 16)] = v
                    else:
                        bmsg[st][1][r, pl.ds((q - 4) * 16, 16)] = v
            scs = [pltpu.async_copy(bmsg[st][p], mouts[p].at[pl.ds(g * K2, K2)],
                                    ssem[st]) for p in range(2)]
            b_issue(st, m + 2)
            for cp in scs:
                cp.wait()

    for st in range(2):
        for p in range(2):
            pltpu.make_async_copy(tabs[p].at[bidx[st][p]], bmb[st][p],
                                  gsem[st][p]).wait()

    et = (t0, t1, t2)
    tabt = (tt0, tt1, tt2)
    moutt = (mt0, mt1, mt2)
    nI3 = 2 * -(-NB3 // (2 * NW))

    def t_issue(st, m):
        g = jnp.minimum(m * NW + w, NB3 - 1)
        off = g * K3
        for p in range(3):
            pltpu.sync_copy(et[p].at[pl.ds(off, K3)], tidx[st][p])
        for p in range(3):
            pltpu.async_copy(tabt[p].at[tidx[st][p]], tmb[st][p], gsem[st][p])

    for st in range(2):
        t_issue(st, st)

    @pl.loop(0, nI3, step=2)
    def _(i):
        for st in range(2):
            m = i + st
            g = jnp.minimum(m * NW + w, NB3 - 1)
            for p in range(3):
                pltpu.make_async_copy(tabt[p].at[tidx[st][p]], tmb[st][p],
                                      gsem[st][p]).wait()
            @pl.loop(0, K3, unroll=4)
            def _(r):
                for q in range(12):
                    a = tmb[st][0][r, pl.ds(q * 16, 16)]
                    b = tmb[st][1][r, pl.ds(q * 16, 16)]
                    cc = tmb[st][2][r, pl.ds(q * 16, 16)]
                    v = jnp.maximum(a + b + cc, 0.0)
                    tmsg[st][q // 4][r, pl.ds((q % 4) * 16, 16)] = v
            scs = [pltpu.async_copy(tmsg[st][p], moutt[p].at[pl.ds(g * K3, K3)],
                                    ssem[st]) for p in range(3)]
            t_issue(st, m + 2)
            for cp in scs:
                cp.wait()

    for st in range(2):
        for p in range(3):
            pltpu.make_async_copy(tabt[p].at[tidx[st][p]], tmb[st][p],
                                  gsem[st][p]).wait()


def _msgs(e0, e1, t0, t1, t2, tb0, tb1, tt0, tt1, tt2):
    f32 = jnp.float32
    return pl.kernel(
        _msg_body,
        out_type=[jax.ShapeDtypeStruct((E2, D), f32),
                  jax.ShapeDtypeStruct((E2, D), f32),
                  jax.ShapeDtypeStruct((E3, D), f32),
                  jax.ShapeDtypeStruct((E3, D), f32),
                  jax.ShapeDtypeStruct((E3, D), f32)],
        mesh=_MESH,
        scratch_types=[
            [[pltpu.VMEM((K2,), jnp.int32) for _ in range(2)] for _ in range(2)],
            [[pltpu.VMEM((K2, 2 * D), f32) for _ in range(2)] for _ in range(2)],
            [[pltpu.VMEM((K2, D), f32) for _ in range(2)] for _ in range(2)],
            [[pltpu.VMEM((K3,), jnp.int32) for _ in range(3)] for _ in range(2)],
            [[pltpu.VMEM((K3, TW), f32) for _ in range(3)] for _ in range(2)],
            [[pltpu.VMEM((K3, D), f32) for _ in range(3)] for _ in range(2)],
            [[pltpu.SemaphoreType.DMA for _ in range(3)] for _ in range(2)],
            [pltpu.SemaphoreType.DMA for _ in range(2)],
        ],
    )(e0, e1, t0, t1, t2, tb0, tb1, tt0, tt1, tt2)


def _conv(e0, e1, t0, t1, t2, tb0, tb1, tt0, tt1, tt2):
    m0, m1, mt0, mt1, mt2 = _msgs(e0, e1, t0, t1, t2, tb0, tb1, tt0, tt1, tt2)
    agg = jnp.zeros((NP, D), jnp.float32)
    agg = agg.at[e0].add(m0).at[e1].add(m1)
    agg = agg.at[t0].add(mt0).at[t1].add(mt1).at[t2].add(mt2)
    return agg


def _tgt_body(ti, h2, out, idxg, rowsg, sem):
    c = lax.axis_index("c")
    s = lax.axis_index("s")
    w = s * NCORE + c
    off = w * 32
    pltpu.sync_copy(ti.at[pl.ds(off, 32)], idxg)
    pltpu.async_copy(h2.at[idxg], rowsg, sem).wait()
    pltpu.sync_copy(rowsg, out.at[pl.ds(off, 32)])


def _tgt_gather(ti, h2p):
    return pl.kernel(
        _tgt_body,
        out_type=jax.ShapeDtypeStruct((1024, 2 * D), jnp.float32),
        mesh=_MESH,
        scratch_types=[
            pltpu.VMEM((32,), jnp.int32),
            pltpu.VMEM((32, 2 * D), jnp.float32),
            pltpu.SemaphoreType.DMA,
        ],
    )(ti, h2p)


_GRID = 16
_R = NP // _GRID


def _tables_body(hp_ref, wb0, wb1, bb, wt0, wt1, wt2, bt,
                 tb0, tb1, tt0, tt1, tt2):
    h = hp_ref[:, :D]
    tb0[...] = jnp.dot(h, wb0[...], preferred_element_type=jnp.float32) + bb[...]
    tb1[...] = jnp.dot(h, wb1[...], preferred_element_type=jnp.float32)
    tt0[...] = jnp.dot(h, wt0[...], preferred_element_type=jnp.float32) + bt[...]
    tt1[...] = jnp.dot(h, wt1[...], preferred_element_type=jnp.float32)
    tt2[...] = jnp.dot(h, wt2[...], preferred_element_type=jnp.float32)


def _tables(hp, wb0, wb1, bb, wt0, wt1, wt2, bt):
    full = lambda shape: pl.BlockSpec(shape, lambda i: (0, 0))
    return pl.pallas_call(
        _tables_body,
        grid=(_GRID,),
        in_specs=[pl.BlockSpec((_R, 2 * D), lambda i: (i, 0)),
                  full((D, 2 * D)), full((D, 2 * D)), full((1, 2 * D)),
                  full((D, TW)), full((D, TW)), full((D, TW)), full((1, TW))],
        out_specs=[pl.BlockSpec((_R, 2 * D), lambda i: (i, 0)),
                   pl.BlockSpec((_R, 2 * D), lambda i: (i, 0)),
                   pl.BlockSpec((_R, TW), lambda i: (i, 0)),
                   pl.BlockSpec((_R, TW), lambda i: (i, 0)),
                   pl.BlockSpec((_R, TW), lambda i: (i, 0))],
        out_shape=[jax.ShapeDtypeStruct((NP, 2 * D), jnp.float32),
                   jax.ShapeDtypeStruct((NP, 2 * D), jnp.float32),
                   jax.ShapeDtypeStruct((NP, TW), jnp.float32),
                   jax.ShapeDtypeStruct((NP, TW), jnp.float32),
                   jax.ShapeDtypeStruct((NP, TW), jnp.float32)],
    )(hp, wb0, wb1, bb, wt0, wt1, wt2, bt)


def _update_core(h, agg, wroot, broot, g, b):
    z = jnp.maximum(jnp.dot(h, wroot[...], preferred_element_type=jnp.float32)
                    + broot[...] + agg, 0.0)
    mu = jnp.mean(z, axis=-1, keepdims=True)
    va = jnp.mean((z - mu) ** 2, axis=-1, keepdims=True)
    return (z - mu) * lax.rsqrt(va + 1e-5) * g[...] + b[...]


def _upd_tab_body(hp_ref, agg_ref, wroot, broot, g, b,
                  wb0, wb1, bb, wt0, wt1, wt2, bt,
                  h1, tb0, tb1, tt0, tt1, tt2):
    hn = _update_core(hp_ref[:, :D], agg_ref[...], wroot, broot, g, b)
    h1[...] = hn
    tb0[...] = jnp.dot(hn, wb0[...], preferred_element_type=jnp.float32) + bb[...]
    tb1[...] = jnp.dot(hn, wb1[...], preferred_element_type=jnp.float32)
    tt0[...] = jnp.dot(hn, wt0[...], preferred_element_type=jnp.float32) + bt[...]
    tt1[...] = jnp.dot(hn, wt1[...], preferred_element_type=jnp.float32)
    tt2[...] = jnp.dot(hn, wt2[...], preferred_element_type=jnp.float32)


def _update_tables(hp, agg, wroot, broot, g, b, wb0, wb1, bb, wt0, wt1, wt2, bt):
    full = lambda shape: pl.BlockSpec(shape, lambda i: (0, 0))
    row = lambda w: pl.BlockSpec((_R, w), lambda i: (i, 0))
    return pl.pallas_call(
        _upd_tab_body,
        grid=(_GRID,),
        in_specs=[row(2 * D), row(D),
                  full((D, D)), full((1, D)), full((1, D)), full((1, D)),
                  full((D, 2 * D)), full((D, 2 * D)), full((1, 2 * D)),
                  full((D, TW)), full((D, TW)), full((D, TW)), full((1, TW))],
        out_specs=[row(D), row(2 * D), row(2 * D), row(TW), row(TW), row(TW)],
        out_shape=[jax.ShapeDtypeStruct((NP, D), jnp.float32),
                   jax.ShapeDtypeStruct((NP, 2 * D), jnp.float32),
                   jax.ShapeDtypeStruct((NP, 2 * D), jnp.float32),
                   jax.ShapeDtypeStruct((NP, TW), jnp.float32),
                   jax.ShapeDtypeStruct((NP, TW), jnp.float32),
                   jax.ShapeDtypeStruct((NP, TW), jnp.float32)],
    )(hp, agg, wroot, broot, g, b, wb0, wb1, bb, wt0, wt1, wt2, bt)


def _upd_final_body(h_ref, agg_ref, wroot, broot, g, b, h2p):
    hn = _update_core(h_ref[...], agg_ref[...], wroot, broot, g, b)
    h2p[...] = jnp.concatenate([hn, jnp.zeros_like(hn)], axis=1)


def _update_final(h, agg, wroot, broot, g, b):
    full = lambda shape: pl.BlockSpec(shape, lambda i: (0, 0))
    row = lambda w: pl.BlockSpec((_R, w), lambda i: (i, 0))
    return pl.pallas_call(
        _upd_final_body,
        grid=(_GRID,),
        in_specs=[row(D), row(D),
                  full((D, D)), full((1, D)), full((1, D)), full((1, D))],
        out_specs=[row(2 * D)],
        out_shape=[jax.ShapeDtypeStruct((NP, 2 * D), jnp.float32)],
    )(h, agg, wroot, broot, g, b)[0]


def _head_body(h_ref, wr_ref, br_ref, lg_ref, lb_ref, wo_ref, bo_ref, o_ref):
    h = h_ref[:, :D]
    for i in range(2):
        h = jnp.dot(h, wr_ref[i], preferred_element_type=jnp.float32) + br_ref[i]
        m = jnp.mean(h, axis=-1, keepdims=True)
        v = jnp.mean((h - m) ** 2, axis=-1, keepdims=True)
        h = (h - m) * jax.lax.rsqrt(v + 1e-5) * lg_ref[i] + lb_ref[i]
        h = jnp.maximum(h, 0.0)
    o_ref[...] = jnp.dot(h, wo_ref[...], preferred_element_type=jnp.float32) + bo_ref[0]


def _head(h_sel, Wr, br, lnr_g, lnr_b, Wout, bout):
    B = h_sel.shape[0]
    return pl.pallas_call(
        _head_body,
        out_shape=jax.ShapeDtypeStruct((B, 1), jnp.float32),
    )(h_sel, Wr, br, lnr_g, lnr_b, Wout, bout)


def _cats(Wbin, bbin, Wter, bter, l):
    Wb = Wbin[l]
    wb0 = jnp.concatenate([Wb[0, :D], Wb[1, :D]], axis=1)
    wb1 = jnp.concatenate([Wb[0, D:], Wb[1, D:]], axis=1)
    bb = jnp.concatenate([bbin[l, 0], bbin[l, 1]])[None]
    Wt = Wter[l]
    zpad = jnp.zeros((D, D), jnp.float32)
    wt = [jnp.concatenate([Wt[0, p * D:(p + 1) * D], Wt[1, p * D:(p + 1) * D],
                           Wt[2, p * D:(p + 1) * D], zpad], axis=1)
          for p in range(3)]
    bt = jnp.concatenate([bter[l, 0], bter[l, 1], bter[l, 2],
                          jnp.zeros((D,), jnp.float32)])[None]
    return wb0, wb1, bb, wt[0], wt[1], wt[2], bt


def kernel(x, edge_index, target_indices, edge_list, emb, Wbin, bbin, Wter, bter,
           Wroot, broot, ln_g, ln_b, Wr, br, lnr_g, lnr_b, Wout, bout):
    x = jnp.ravel(x)
    ti = jnp.ravel(target_indices)
    e0, e1 = edge_index[0], edge_index[1]
    t0, t1, t2 = edge_list[0], edge_list[1], edge_list[2]

    xpad = jnp.pad(x, (0, NP - N))
    embp = jnp.pad(emb, ((0, 0), (0, D)))
    hp = _emb_gather(xpad, embp)

    c0 = _cats(Wbin, bbin, Wter, bter, 0)
    c1 = _cats(Wbin, bbin, Wter, bter, 1)

    T0 = _tables(hp, *c0)
    agg0 = _conv(e0, e1, t0, t1, t2, *T0)
    h1, *T1 = _update_tables(hp, agg0, Wroot[0], broot[0][None],
                             ln_g[0][None], ln_b[0][None], *c1)
    agg1 = _conv(e0, e1, t0, t1, t2, *T1)
    h2p = _update_final(h1, agg1, Wroot[1], broot[1][None],
                        ln_g[1][None], ln_b[1][None])
    hs = _tgt_gather(ti, h2p)
    return _head(hs, Wr, br, lnr_g, lnr_b, Wout, bout)


# speedup vs baseline: 1.1317x; 1.1317x over previous
import functools

import jax
import jax.numpy as jnp
from jax import lax
from jax.experimental import pallas as pl
from jax.experimental.pallas import tpu as pltpu
from jax.experimental.pallas import tpu_sc as plsc

N = 50000
NP = 50176
D = 64
L = 2
E2 = 800000
E3 = 200000
NCORE = 2
NSUB = 16
NW = NCORE * NSUB

K2 = 32
K3 = 32
KS = 32
NB2 = E2 // K2
NB3 = E3 // K3
HALF = N // 2
HPAD = 25088
DUMB = HALF
ZR = 56
TW = 4 * D

_MESH = plsc.VectorSubcoreMesh(
    core_axis_name="c", subcore_axis_name="s", num_cores=NCORE, num_subcores=NSUB)

_ROWS_W = NP // NW
_GB = 128
_GNB = 12
_GT = _ROWS_W - _GNB * _GB


def _emb_body(xpad, emb, out, idxb, rows, idxt, rowst, sem):
    c = lax.axis_index("c")
    s = lax.axis_index("s")
    w = s * NCORE + c
    base = w * _ROWS_W
    @pl.loop(0, _GNB)
    def _(k):
        off = base + k * _GB
        pltpu.sync_copy(xpad.at[pl.ds(off, _GB)], idxb)
        pltpu.async_copy(emb.at[idxb], rows, sem).wait()
        pltpu.sync_copy(rows, out.at[pl.ds(off, _GB)])
    offt = base + _GNB * _GB
    pltpu.sync_copy(xpad.at[pl.ds(offt, _GT)], idxt)
    pltpu.async_copy(emb.at[idxt], rowst, sem).wait()
    pltpu.sync_copy(rowst, out.at[pl.ds(offt, _GT)])


def _emb_gather(xpad, embp):
    return pl.kernel(
        _emb_body,
        out_type=jax.ShapeDtypeStruct((NP, 2 * D), jnp.float32),
        mesh=_MESH,
        scratch_types=[
            pltpu.VMEM((_GB,), jnp.int32),
            pltpu.VMEM((_GB, 2 * D), jnp.float32),
            pltpu.VMEM((_GT,), jnp.int32),
            pltpu.VMEM((_GT, 2 * D), jnp.float32),
            pltpu.SemaphoreType.DMA,
        ],
    )(xpad, embp)


def _msg_body(e0, e1, t0, t1, t2, tb0, tb1, tt0, tt1, tt2,
              m0, m1, mt0, mt1, mt2,
              bidx, bmb, bmsg, tidx, tmb, tmsg, gsem, ssem):
    c = lax.axis_index("c")
    s = lax.axis_index("s")
    w = s * NCORE + c

    eb = (e0, e1)
    tabs = (tb0, tb1)
    mouts = (m0, m1)
    nI2 = 2 * -(-NB2 // (2 * NW))

    def b_issue(st, m):
        g = jnp.minimum(m * NW + w, NB2 - 1)
        off = g * K2
        for p in range(2):
            pltpu.sync_copy(eb[p].at[pl.ds(off, K2)], bidx[st][p])
        for p in range(2):
            pltpu.async_copy(tabs[p].at[bidx[st][p]], bmb[st][p], gsem[st][p])

    for st in range(2):
        b_issue(st, st)

    @pl.loop(0, nI2, step=2)
    def _(i):
        for st in range(2):
            m = i + st
            g = jnp.minimum(m * NW + w, NB2 - 1)
            for p in range(2):
                pltpu.make_async_copy(tabs[p].at[bidx[st][p]], bmb[st][p],
                                      gsem[st][p]).wait()
            @pl.loop(0, K2)
            def _(r):
                for q in range(8):
                    a = bmb[st][0][r, pl.ds(q * 16, 16)]
                    b = bmb[st][1][r, pl.ds(q * 16, 16)]
                    v = jnp.maximum(a + b, 0.0)
                    if q < 4:
                        bmsg[st][0][r, pl.ds(q * 16, 16)] = v
                    else:
                        bmsg[st][1][r, pl.ds((q - 4) * 16, 16)] = v
            scs = [pltpu.async_copy(bmsg[st][p], mouts[p].at[pl.ds(g * K2, K2)],
                                    ssem[st]) for p in range(2)]
            b_issue(st, m + 2)
            for cp in scs:
                cp.wait()

    for st in range(2):
        for p in range(2):
            pltpu.make_async_copy(tabs[p].at[bidx[st][p]], bmb[st][p],
                                  gsem[st][p]).wait()

    et = (t0, t1, t2)
    tabt = (tt0, tt1, tt2)
    moutt = (mt0, mt1, mt2)
    nI3 = 2 * -(-NB3 // (2 * NW))

    def t_issue(st, m):
        g = jnp.minimum(m * NW + w, NB3 - 1)
        off = g * K3
        for p in range(3):
            pltpu.sync_copy(et[p].at[pl.ds(off, K3)], tidx[st][p])
        for p in range(3):
            pltpu.async_copy(tabt[p].at[tidx[st][p]], tmb[st][p], gsem[st][p])

    for st in range(2):
        t_issue(st, st)

    @pl.loop(0, nI3, step=2)
    def _(i):
        for st in range(2):
            m = i + st
            g = jnp.minimum(m * NW + w, NB3 - 1)
            for p in range(3):
                pltpu.make_async_copy(tabt[p].at[tidx[st][p]], tmb[st][p],
                                      gsem[st][p]).wait()
            @pl.loop(0, K3)
            def _(r):
                for q in range(12):
                    a = tmb[st][0][r, pl.ds(q * 16, 16)]
                    b = tmb[st][1][r, pl.ds(q * 16, 16)]
                    cc = tmb[st][2][r, pl.ds(q * 16, 16)]
                    v = jnp.maximum(a + b + cc, 0.0)
                    tmsg[st][q // 4][r, pl.ds((q % 4) * 16, 16)] = v
            scs = [pltpu.async_copy(tmsg[st][p], moutt[p].at[pl.ds(g * K3, K3)],
                                    ssem[st]) for p in range(3)]
            t_issue(st, m + 2)
            for cp in scs:
                cp.wait()

    for st in range(2):
        for p in range(3):
            pltpu.make_async_copy(tabt[p].at[tidx[st][p]], tmb[st][p],
                                  gsem[st][p]).wait()


def _msgs(e0, e1, t0, t1, t2, tb0, tb1, tt0, tt1, tt2):
    f32 = jnp.float32
    return pl.kernel(
        _msg_body,
        out_type=[jax.ShapeDtypeStruct((E2, D), f32),
                  jax.ShapeDtypeStruct((E2, D), f32),
                  jax.ShapeDtypeStruct((E3, D), f32),
                  jax.ShapeDtypeStruct((E3, D), f32),
                  jax.ShapeDtypeStruct((E3, D), f32)],
        mesh=_MESH,
        scratch_types=[
            [[pltpu.VMEM((K2,), jnp.int32) for _ in range(2)] for _ in range(2)],
            [[pltpu.VMEM((K2, 2 * D), f32) for _ in range(2)] for _ in range(2)],
            [[pltpu.VMEM((K2, D), f32) for _ in range(2)] for _ in range(2)],
            [[pltpu.VMEM((K3,), jnp.int32) for _ in range(3)] for _ in range(2)],
            [[pltpu.VMEM((K3, TW), f32) for _ in range(3)] for _ in range(2)],
            [[pltpu.VMEM((K3, D), f32) for _ in range(3)] for _ in range(2)],
            [[pltpu.SemaphoreType.DMA for _ in range(3)] for _ in range(2)],
            [pltpu.SemaphoreType.DMA for _ in range(2)],
        ],
    )(e0, e1, t0, t1, t2, tb0, tb1, tt0, tt1, tt2)


def _conv(e0, e1, t0, t1, t2, tb0, tb1, tt0, tt1, tt2):
    m0, m1, mt0, mt1, mt2 = _msgs(e0, e1, t0, t1, t2, tb0, tb1, tt0, tt1, tt2)
    agg = jnp.zeros((NP, D), jnp.float32)
    agg = agg.at[e0].add(m0).at[e1].add(m1)
    agg = agg.at[t0].add(mt0).at[t1].add(mt1).at[t2].add(mt2)
    return agg


def _tgt_body(ti, h2, out, idxg, rowsg, sem):
    c = lax.axis_index("c")
    s = lax.axis_index("s")
    w = s * NCORE + c
    off = w * 32
    pltpu.sync_copy(ti.at[pl.ds(off, 32)], idxg)
    pltpu.async_copy(h2.at[idxg], rowsg, sem).wait()
    pltpu.sync_copy(rowsg, out.at[pl.ds(off, 32)])


def _tgt_gather(ti, h2p):
    return pl.kernel(
        _tgt_body,
        out_type=jax.ShapeDtypeStruct((1024, 2 * D), jnp.float32),
        mesh=_MESH,
        scratch_types=[
            pltpu.VMEM((32,), jnp.int32),
            pltpu.VMEM((32, 2 * D), jnp.float32),
            pltpu.SemaphoreType.DMA,
        ],
    )(ti, h2p)


_GRID = 16
_R = NP // _GRID


def _tables_body(hp_ref, wb0, wb1, bb, wt0, wt1, wt2, bt,
                 tb0, tb1, tt0, tt1, tt2):
    h = hp_ref[:, :D]
    tb0[...] = jnp.dot(h, wb0[...], preferred_element_type=jnp.float32) + bb[...]
    tb1[...] = jnp.dot(h, wb1[...], preferred_element_type=jnp.float32)
    tt0[...] = jnp.dot(h, wt0[...], preferred_element_type=jnp.float32) + bt[...]
    tt1[...] = jnp.dot(h, wt1[...], preferred_element_type=jnp.float32)
    tt2[...] = jnp.dot(h, wt2[...], preferred_element_type=jnp.float32)


def _tables(hp, wb0, wb1, bb, wt0, wt1, wt2, bt):
    full = lambda shape: pl.BlockSpec(shape, lambda i: (0, 0))
    return pl.pallas_call(
        _tables_body,
        grid=(_GRID,),
        in_specs=[pl.BlockSpec((_R, 2 * D), lambda i: (i, 0)),
                  full((D, 2 * D)), full((D, 2 * D)), full((1, 2 * D)),
                  full((D, TW)), full((D, TW)), full((D, TW)), full((1, TW))],
        out_specs=[pl.BlockSpec((_R, 2 * D), lambda i: (i, 0)),
                   pl.BlockSpec((_R, 2 * D), lambda i: (i, 0)),
                   pl.BlockSpec((_R, TW), lambda i: (i, 0)),
                   pl.BlockSpec((_R, TW), lambda i: (i, 0)),
                   pl.BlockSpec((_R, TW), lambda i: (i, 0))],
        out_shape=[jax.ShapeDtypeStruct((NP, 2 * D), jnp.float32),
                   jax.ShapeDtypeStruct((NP, 2 * D), jnp.float32),
                   jax.ShapeDtypeStruct((NP, TW), jnp.float32),
                   jax.ShapeDtypeStruct((NP, TW), jnp.float32),
                   jax.ShapeDtypeStruct((NP, TW), jnp.float32)],
    )(hp, wb0, wb1, bb, wt0, wt1, wt2, bt)


def _update_core(h, agg, wroot, broot, g, b):
    z = jnp.maximum(jnp.dot(h, wroot[...], preferred_element_type=jnp.float32)
                    + broot[...] + agg, 0.0)
    mu = jnp.mean(z, axis=-1, keepdims=True)
    va = jnp.mean((z - mu) ** 2, axis=-1, keepdims=True)
    return (z - mu) * lax.rsqrt(va + 1e-5) * g[...] + b[...]


def _upd_tab_body(hp_ref, agg_ref, wroot, broot, g, b,
                  wb0, wb1, bb, wt0, wt1, wt2, bt,
                  h1, tb0, tb1, tt0, tt1, tt2):
    hn = _update_core(hp_ref[:, :D], agg_ref[...], wroot, broot, g, b)
    h1[...] = hn
    tb0[...] = jnp.dot(hn, wb0[...], preferred_element_type=jnp.float32) + bb[...]
    tb1[...] = jnp.dot(hn, wb1[...], preferred_element_type=jnp.float32)
    tt0[...] = jnp.dot(hn, wt0[...], preferred_element_type=jnp.float32) + bt[...]
    tt1[...] = jnp.dot(hn, wt1[...], preferred_element_type=jnp.float32)
    tt2[...] = jnp.dot(hn, wt2[...], preferred_element_type=jnp.float32)


def _update_tables(hp, agg, wroot, broot, g, b, wb0, wb1, bb, wt0, wt1, wt2, bt):
    full = lambda shape: pl.BlockSpec(shape, lambda i: (0, 0))
    row = lambda w: pl.BlockSpec((_R, w), lambda i: (i, 0))
    return pl.pallas_call(
        _upd_tab_body,
        grid=(_GRID,),
        in_specs=[row(2 * D), row(D),
                  full((D, D)), full((1, D)), full((1, D)), full((1, D)),
                  full((D, 2 * D)), full((D, 2 * D)), full((1, 2 * D)),
                  full((D, TW)), full((D, TW)), full((D, TW)), full((1, TW))],
        out_specs=[row(D), row(2 * D), row(2 * D), row(TW), row(TW), row(TW)],
        out_shape=[jax.ShapeDtypeStruct((NP, D), jnp.float32),
                   jax.ShapeDtypeStruct((NP, 2 * D), jnp.float32),
                   jax.ShapeDtypeStruct((NP, 2 * D), jnp.float32),
                   jax.ShapeDtypeStruct((NP, TW), jnp.float32),
                   jax.ShapeDtypeStruct((NP, TW), jnp.float32),
                   jax.ShapeDtypeStruct((NP, TW), jnp.float32)],
    )(hp, agg, wroot, broot, g, b, wb0, wb1, bb, wt0, wt1, wt2, bt)


def _upd_final_body(h_ref, agg_ref, wroot, broot, g, b, h2p):
    hn = _update_core(h_ref[...], agg_ref[...], wroot, broot, g, b)
    h2p[...] = jnp.concatenate([hn, jnp.zeros_like(hn)], axis=1)


def _update_final(h, agg, wroot, broot, g, b):
    full = lambda shape: pl.BlockSpec(shape, lambda i: (0, 0))
    row = lambda w: pl.BlockSpec((_R, w), lambda i: (i, 0))
    return pl.pallas_call(
        _upd_final_body,
        grid=(_GRID,),
        in_specs=[row(D), row(D),
                  full((D, D)), full((1, D)), full((1, D)), full((1, D))],
        out_specs=[row(2 * D)],
        out_shape=[jax.ShapeDtypeStruct((NP, 2 * D), jnp.float32)],
    )(h, agg, wroot, broot, g, b)[0]


def _head_body(h_ref, wr_ref, br_ref, lg_ref, lb_ref, wo_ref, bo_ref, o_ref):
    h = h_ref[:, :D]
    for i in range(2):
        h = jnp.dot(h, wr_ref[i], preferred_element_type=jnp.float32) + br_ref[i]
        m = jnp.mean(h, axis=-1, keepdims=True)
        v = jnp.mean((h - m) ** 2, axis=-1, keepdims=True)
        h = (h - m) * jax.lax.rsqrt(v + 1e-5) * lg_ref[i] + lb_ref[i]
        h = jnp.maximum(h, 0.0)
    o_ref[...] = jnp.dot(h, wo_ref[...], preferred_element_type=jnp.float32) + bo_ref[0]


def _head(h_sel, Wr, br, lnr_g, lnr_b, Wout, bout):
    B = h_sel.shape[0]
    return pl.pallas_call(
        _head_body,
        out_shape=jax.ShapeDtypeStruct((B, 1), jnp.float32),
    )(h_sel, Wr, br, lnr_g, lnr_b, Wout, bout)


def _cats(Wbin, bbin, Wter, bter, l):
    Wb = Wbin[l]
    wb0 = jnp.concatenate([Wb[0, :D], Wb[1, :D]], axis=1)
    wb1 = jnp.concatenate([Wb[0, D:], Wb[1, D:]], axis=1)
    bb = jnp.concatenate([bbin[l, 0], bbin[l, 1]])[None]
    Wt = Wter[l]
    zpad = jnp.zeros((D, D), jnp.float32)
    wt = [jnp.concatenate([Wt[0, p * D:(p + 1) * D], Wt[1, p * D:(p + 1) * D],
                           Wt[2, p * D:(p + 1) * D], zpad], axis=1)
          for p in range(3)]
    bt = jnp.concatenate([bter[l, 0], bter[l, 1], bter[l, 2],
                          jnp.zeros((D,), jnp.float32)])[None]
    return wb0, wb1, bb, wt[0], wt[1], wt[2], bt


def kernel(x, edge_index, target_indices, edge_list, emb, Wbin, bbin, Wter, bter,
           Wroot, broot, ln_g, ln_b, Wr, br, lnr_g, lnr_b, Wout, bout):
    x = jnp.ravel(x)
    ti = jnp.ravel(target_indices)
    e0, e1 = edge_index[0], edge_index[1]
    t0, t1, t2 = edge_list[0], edge_list[1], edge_list[2]

    xpad = jnp.pad(x, (0, NP - N))
    embp = jnp.pad(emb, ((0, 0), (0, D)))
    hp = _emb_gather(xpad, embp)

    c0 = _cats(Wbin, bbin, Wter, bter, 0)
    c1 = _cats(Wbin, bbin, Wter, bter, 1)

    T0 = _tables(hp, *c0)
    agg0 = _conv(e0, e1, t0, t1, t2, *T0)
    h1, *T1 = _update_tables(hp, agg0, Wroot[0], broot[0][None],
                             ln_g[0][None], ln_b[0][None], *c1)
    agg1 = _conv(e0, e1, t0, t1, t2, *T1)
    h2p = _update_final(h1, agg1, Wroot[1], broot[1][None],
                        ln_g[1][None], ln_b[1][None])
    hs = _tgt_gather(ti, h2p)
    return _head(hs, Wr, br, lnr_g, lnr_b, Wout, bout)
